# baseline (speedup 1.0000x reference)
import jax
import jax.numpy as jnp
import numpy as np
from functools import partial
from jax.experimental import pallas as pl
from jax.sharding import Mesh, PartitionSpec as P

try:
    from jax import shard_map as _shard_map
except ImportError:
    from jax.experimental.shard_map import shard_map as _shard_map

_BATCH = 16384
_STATE_DIM = 128
_NUM_STEPS = 50
_NUM_PRIMS = 16
_TILE = 2048


def _exec_kernel(program_ref, step_emb_ref, lib_W_ref, lib_b_ref,
                 state_ref, out_ref):
    p = jax.nn.softmax(program_ref[:], axis=-1)
    w = jnp.dot(p, lib_W_ref[:], preferred_element_type=jnp.float32)
    b = jnp.dot(p, lib_b_ref[:], preferred_element_type=jnp.float32)
    c = step_emb_ref[:] * w + b
    x = state_ref[:]
    for i in range(_NUM_STEPS):
        x = jnp.tanh(x * w[i] + c[i])
    out_ref[:] = x


def _run_shard(state_s, program, step_emb, lib_W, lib_b):
    rows = state_s.shape[0]
    tile = min(_TILE, rows)
    grid = (rows // tile,)
    return pl.pallas_call(
        _exec_kernel,
        grid=grid,
        in_specs=[
            pl.BlockSpec((_NUM_STEPS, _NUM_PRIMS), lambda i: (0, 0)),
            pl.BlockSpec((_NUM_STEPS, _STATE_DIM), lambda i: (0, 0)),
            pl.BlockSpec((_NUM_PRIMS, _STATE_DIM), lambda i: (0, 0)),
            pl.BlockSpec((_NUM_PRIMS, _STATE_DIM), lambda i: (0, 0)),
            pl.BlockSpec((tile, _STATE_DIM), lambda i: (i, 0)),
        ],
        out_specs=pl.BlockSpec((tile, _STATE_DIM), lambda i: (i, 0)),
        out_shape=jax.ShapeDtypeStruct((rows, _STATE_DIM), jnp.float32),
    )(program, step_emb, lib_W, lib_b, state_s)


def kernel(state, program, step_emb, lib_W, lib_b):
    devs = jax.devices()
    n = len(devs)
    while n > 1 and _BATCH % n != 0:
        n -= 1
    mesh = Mesh(np.array(devs[:n]), ("d",))
    run = _shard_map(
        _run_shard,
        mesh=mesh,
        in_specs=(P("d", None), P(None, None), P(None, None),
                  P(None, None), P(None, None)),
        out_specs=P("d", None),
        check_vma=False,
    )
    final = run(state, program, step_emb, lib_W, lib_b)
    return (final, program)


# speedup vs baseline: 4.2268x; 4.2268x over previous
import functools
import jax
import jax.numpy as jnp
from jax import lax
from jax.experimental import pallas as pl
from jax.experimental.pallas import tpu as pltpu
from jax.experimental.pallas import tpu_sc as plsc

_BATCH = 16384
_D = 128
_S = 50
_P = 16
_TILE = 2048

_SC_ROWS = 2048
_TC_ROWS = _BATCH - _SC_ROWS
_NW = 32
_RPT = _SC_ROWS // _NW
_CHUNKS = _D // 16


def _coef_kernel(program_ref, step_emb_ref, lib_W_ref, lib_b_ref,
                 w_ref, c_ref, w2_ref, c2_ref):
    p = jax.nn.softmax(program_ref[:], axis=-1)
    w = jnp.dot(p, lib_W_ref[:], preferred_element_type=jnp.float32)
    b = jnp.dot(p, lib_b_ref[:], preferred_element_type=jnp.float32)
    c = step_emb_ref[:] * w + b
    w_ref[:] = w
    c_ref[:] = c
    w2_ref[:] = 2.0 * w
    c2_ref[:] = 2.0 * c


def _coefs(program, step_emb, lib_W, lib_b):
    out = jax.ShapeDtypeStruct((_S, _D), jnp.float32)
    return pl.pallas_call(
        _coef_kernel,
        out_shape=(out, out, out, out),
    )(program, step_emb, lib_W, lib_b)


def _tc_kernel(w_ref, c_ref, state_ref, out_ref):
    x = state_ref[:]
    for i in range(_S):
        x = jnp.tanh(x * w_ref[i] + c_ref[i])
    out_ref[:] = x


def _tc_run(state_tc, w, c):
    grid = (_TC_ROWS // _TILE,)
    return pl.pallas_call(
        _tc_kernel,
        grid=grid,
        in_specs=[
            pl.BlockSpec((_S, _D), lambda i: (0, 0)),
            pl.BlockSpec((_S, _D), lambda i: (0, 0)),
            pl.BlockSpec((_TILE, _D), lambda i: (i, 0)),
        ],
        out_specs=pl.BlockSpec((_TILE, _D), lambda i: (i, 0)),
        out_shape=jax.ShapeDtypeStruct((_TC_ROWS, _D), jnp.float32),
    )(w, c, state_tc)


def _sc_run(state_sc, w2, c2):
    mesh = plsc.VectorSubcoreMesh(core_axis_name="c", subcore_axis_name="s",
                                  num_cores=2, num_subcores=16)

    @functools.partial(
        pl.kernel,
        mesh=mesh,
        out_type=jax.ShapeDtypeStruct((_SC_ROWS, _D), jnp.float32),
        scratch_types=[
            pltpu.VMEM((_S, _D), jnp.float32),
            pltpu.VMEM((_S, _D), jnp.float32),
            pltpu.VMEM((_RPT, _D), jnp.float32),
        ],
    )
    def sc_k(state_hbm, w2_hbm, c2_hbm, out_hbm, w2_v, c2_v, x_v):
        wid = lax.axis_index("s") * 2 + lax.axis_index("c")
        base = wid * _RPT
        pltpu.sync_copy(w2_hbm, w2_v)
        pltpu.sync_copy(c2_hbm, c2_v)
        pltpu.sync_copy(state_hbm.at[pl.ds(base, _RPT)], x_v)

        def step_body(s, carry):
            for k in range(_CHUNKS):
                wv = w2_v[s, pl.ds(k * 16, 16)]
                cv = c2_v[s, pl.ds(k * 16, 16)]

                def row_body(r, _, k=k, wv=wv, cv=cv):
                    x = x_v[r, pl.ds(k * 16, 16)]
                    e = jnp.exp(x * wv + cv)
                    x_v[r, pl.ds(k * 16, 16)] = 1.0 - 2.0 / (1.0 + e)
                    return 0

                lax.fori_loop(0, _RPT, row_body, 0, unroll=8)
            return carry

        lax.fori_loop(0, _S, step_body, 0)
        pltpu.sync_copy(x_v, out_hbm.at[pl.ds(base, _RPT)])

    return sc_k(state_sc, w2, c2)


def kernel(state, program, step_emb, lib_W, lib_b):
    w, c, w2, c2 = _coefs(program, step_emb, lib_W, lib_b)
    final_sc = _sc_run(state[_TC_ROWS:], w2, c2)
    final_tc = _tc_run(state[:_TC_ROWS], w, c)
    final = jnp.concatenate([final_tc, final_sc], axis=0)
    return (final, program)
